# Initial kernel scaffold; baseline (speedup 1.0000x reference)
import jax
import jax.numpy as jnp
from jax.experimental import pallas as pl


def kernel(x, edge_index, edge_attr, batch, params):
    raise NotImplementedError("write your pallas kernel here")


# speedup vs baseline: 2.5068x; 2.5068x over previous
import functools

import jax
import jax.numpy as jnp
from jax import lax
from jax.experimental import pallas as pl
from jax.experimental.pallas import tpu as pltpu
from jax.experimental.pallas import tpu_sc as plsc

SLOPE = 0.01
LANES = 16
NUM_CORES = 2
NUM_SUBCORES = 16
CHUNK = 80
ZROWS = 104


def _lrelu(t):
    return jnp.where(t >= 0, t, t * SLOPE)


def _edge_embed_body(ea_ref, w_ref, b_ref, out_ref):
    out_ref[...] = (
        jnp.dot(ea_ref[...], w_ref[...], preferred_element_type=jnp.float32)
        + b_ref[...]
    )


def _edge_embed(edge_attr, w, b):
    E, K = edge_attr.shape
    D = w.shape[1]
    BLK = 4000
    grid = (E // BLK,)
    return pl.pallas_call(
        _edge_embed_body,
        grid=grid,
        in_specs=[
            pl.BlockSpec((BLK, K), lambda i: (i, 0)),
            pl.BlockSpec((K, D), lambda i: (0, 0)),
            pl.BlockSpec((1, D), lambda i: (0, 0)),
        ],
        out_specs=pl.BlockSpec((BLK, D), lambda i: (i, 0)),
        out_shape=jax.ShapeDtypeStruct((E, D), jnp.float32),
    )(edge_attr, w, b.reshape(1, D))


def _sc_aggregate(h, e, src, dst):
    N, D = h.shape
    E = src.shape[0]
    NW = NUM_CORES * NUM_SUBCORES
    epw = E // NW
    n_chunks = epw // CHUNK
    rpt = (N // NUM_SUBCORES) // 8 * 8
    tail = N - NUM_SUBCORES * rpt
    nvec = D // LANES

    mesh = plsc.VectorSubcoreMesh(core_axis_name="c", subcore_axis_name="s")

    @functools.partial(
        pl.kernel,
        mesh=mesh,
        out_type=jax.ShapeDtypeStruct((NUM_CORES, N, D), jnp.float32),
        scratch_types=[
            pltpu.VMEM((CHUNK,), jnp.int32),
            pltpu.VMEM((CHUNK,), jnp.int32),
            pltpu.VMEM((CHUNK, D), jnp.float32),
            pltpu.VMEM((CHUNK, D), jnp.float32),
            pltpu.VMEM((ZROWS, D), jnp.float32),
            pltpu.VMEM_SHARED((N, D), jnp.float32),
            pltpu.SemaphoreType.DMA,
        ],
    )
    def k(h_hbm, e_hbm, src_hbm, dst_hbm, out_hbm,
          src_v, dst_v, rows_v, e_v, z_v, acc_sh, sem):
        c = lax.axis_index("c")
        s = lax.axis_index("s")
        wid = c * NUM_SUBCORES + s

        def zrow(i, carry):
            for j in range(nvec):
                z_v[i, pl.ds(j * LANES, LANES)] = jnp.zeros((LANES,), jnp.float32)
            return carry
        lax.fori_loop(0, ZROWS, zrow, 0)
        for r in range(rpt // ZROWS):
            pltpu.sync_copy(z_v, acc_sh.at[pl.ds(s * rpt + r * ZROWS, ZROWS), :])
        if tail:
            @pl.when(s == NUM_SUBCORES - 1)
            def _():
                pltpu.sync_copy(z_v.at[pl.ds(0, tail), :],
                                acc_sh.at[pl.ds(NUM_SUBCORES * rpt, tail), :])
        plsc.subcore_barrier()

        base0 = wid * epw

        def chunk_body(i, carry):
            base = base0 + i * CHUNK
            pltpu.sync_copy(src_hbm.at[pl.ds(base, CHUNK)], src_v)
            pltpu.sync_copy(dst_hbm.at[pl.ds(base, CHUNK)], dst_v)
            pltpu.async_copy(h_hbm.at[src_v], rows_v, sem).wait()
            pltpu.sync_copy(e_hbm.at[pl.ds(base, CHUNK), :], e_v)

            def crow(r, cc):
                for j in range(nvec):
                    sl = pl.ds(j * LANES, LANES)
                    rows_v[r, sl] = jnp.maximum(rows_v[r, sl] + e_v[r, sl], 0.0)
                return cc
            lax.fori_loop(0, CHUNK, crow, 0)

            pltpu.sync_copy(rows_v, acc_sh.at[dst_v], add=True)
            return carry
        lax.fori_loop(0, n_chunks, chunk_body, 0)

        plsc.subcore_barrier()
        pltpu.sync_copy(
            acc_sh.at[pl.ds(s * rpt, rpt), :],
            out_hbm.at[c, pl.ds(s * rpt, rpt), :],
        )
        if tail:
            @pl.when(s == NUM_SUBCORES - 1)
            def _():
                pltpu.sync_copy(
                    acc_sh.at[pl.ds(NUM_SUBCORES * rpt, tail), :],
                    out_hbm.at[c, pl.ds(NUM_SUBCORES * rpt, tail), :],
                )

    return k(h, e, src, dst)


def _dense_body(nlrelu, h_ref, a_ref, w1_ref, b1_ref, g_ref, bt_ref,
                w2_ref, b2_ref, out_ref):
    x = h_ref[...] + a_ref[0] + a_ref[1]
    t = jnp.dot(x, w1_ref[...], preferred_element_type=jnp.float32) + b1_ref[...]
    mean = jnp.mean(t, axis=0, keepdims=True)
    var = jnp.mean((t - mean) ** 2, axis=0, keepdims=True)
    t = (t - mean) * lax.rsqrt(var + 1e-5) * g_ref[...] + bt_ref[...]
    t = _lrelu(t)
    t = jnp.dot(t, w2_ref[...], preferred_element_type=jnp.float32) + b2_ref[...]
    for _ in range(nlrelu):
        t = _lrelu(t)
    out_ref[...] = t


def _dense(h, agg2, w1, b1, gamma, beta, w2, b2, nlrelu):
    N, D = h.shape
    H = w1.shape[1]
    return pl.pallas_call(
        functools.partial(_dense_body, nlrelu),
        out_shape=jax.ShapeDtypeStruct((N, H), jnp.float32),
    )(h, agg2, w1, b1.reshape(1, H), gamma.reshape(1, H), beta.reshape(1, H),
      w2, b2.reshape(1, H))


def _head_body(h_ref, wr_ref, br_ref, we_ref, be_ref, out_ref):
    t = jnp.dot(h_ref[...], wr_ref[...], preferred_element_type=jnp.float32)
    t = _lrelu(t + br_ref[...])
    out_ref[...] = (
        jnp.dot(t, we_ref[...], preferred_element_type=jnp.float32) + be_ref[...]
    )


def _head(h, wr, br, wend, bend):
    N, D = h.shape
    R = wr.shape[1]
    BLK = 1000
    return pl.pallas_call(
        _head_body,
        grid=(N // BLK,),
        in_specs=[
            pl.BlockSpec((BLK, D), lambda i: (i, 0)),
            pl.BlockSpec((D, R), lambda i: (0, 0)),
            pl.BlockSpec((1, R), lambda i: (0, 0)),
            pl.BlockSpec((R, 1), lambda i: (0, 0)),
            pl.BlockSpec((1, 1), lambda i: (0, 0)),
        ],
        out_specs=pl.BlockSpec((BLK, 1), lambda i: (i, 0)),
        out_shape=jax.ShapeDtypeStruct((N, 1), jnp.float32),
    )(h, wr, br.reshape(1, R), wend, bend.reshape(1, 1))


def kernel(x, edge_index, edge_attr, batch, params):
    src = edge_index[0]
    dst = edge_index[1]
    h = x
    num_layers = 3
    for l in range(num_layers):
        e = _edge_embed(edge_attr, params['We_%d' % l], params['be_%d' % l])
        agg2 = _sc_aggregate(h, e, src, dst)
        h = _dense(h, agg2,
                   params['W1_%d' % l], params['b1_%d' % l],
                   params['gamma_%d' % l], params['beta_%d' % l],
                   params['W2_%d' % l], params['b2_%d' % l],
                   nlrelu=2 if l < num_layers - 1 else 1)
    return _head(h, params['Wr'], params['br'], params['Wend'], params['bend'])
